# baseline (speedup 1.0000x reference)
import functools

import jax
import jax.numpy as jnp
from jax import lax
from jax.experimental import pallas as pl
from jax.experimental.pallas import tpu as pltpu
from jax.experimental.pallas import tpu_sc as plsc

NC = 2
NS = 16
NW = NC * NS
LANES = 16
CHUNK = 128
NBUF = 2


@functools.cache
def _build_sc_call(C, H, W, N):
    TCH = (N + CHUNK - 1) // CHUNK
    TAIL = N - (TCH - 1) * CHUNK
    if TAIL == CHUNK:
        TAIL = 0
    NCH_LO = TCH // NW
    REM = TCH - NCH_LO * NW
    WTAIL = (TCH - 1) % NW
    KTAIL = (TCH - 1) // NW
    assert TAIL % LANES == 0 and NCH_LO >= NBUF

    mesh = plsc.VectorSubcoreMesh(core_axis_name="c", subcore_axis_name="s")

    def vmem(shape, dtype):
        return [pltpu.VMEM(shape, dtype) for _ in range(NBUF)]

    @functools.partial(
        pl.kernel,
        out_type=jax.ShapeDtypeStruct((N, C), jnp.float32),
        mesh=mesh,
        scratch_types=[
            vmem((CHUNK, 2), jnp.float32),
            vmem((CHUNK,), jnp.int32),
            vmem((CHUNK,), jnp.int32),
            vmem((CHUNK,), jnp.int32),
            vmem((CHUNK,), jnp.int32),
            vmem((CHUNK,), jnp.float32),
            vmem((CHUNK,), jnp.float32),
            vmem((CHUNK,), jnp.float32),
            vmem((CHUNK,), jnp.float32),
            vmem((CHUNK, C), jnp.float32),
            vmem((CHUNK, C), jnp.float32),
            vmem((CHUNK, C), jnp.float32),
            vmem((CHUNK, C), jnp.float32),
            vmem((CHUNK, C), jnp.float32),
            [pltpu.SemaphoreType.DMA for _ in range(NBUF)],
            [pltpu.SemaphoreType.DMA for _ in range(NBUF)],
            [pltpu.SemaphoreType.DMA for _ in range(NBUF)],
        ],
        compiler_params=pltpu.CompilerParams(needs_layout_passes=False,
                                             use_tc_tiling_on_sc=False),
    )
    def sc_sample(table, verts, out, vbuf, i00, i01, i10, i11,
                  w00, w01, w10, w11, v00, v01, v10, v11, outb,
                  vsem, gsem, osem):
        cid = lax.axis_index("c")
        sid = lax.axis_index("s")
        wid = sid * NC + cid
        nch = NCH_LO + jnp.where(wid < REM, 1, 0)

        iota = lax.broadcasted_iota(jnp.int32, (LANES,), 0)
        zeros16 = jnp.zeros((LANES,), jnp.int32)
        ones16 = zeros16 + 1

        def rbase(ci):
            return (wid + ci * NW) * CHUNK

        def fetch_verts(ci, s):
            pltpu.async_copy(verts.at[pl.ds(rbase(ci), CHUNK)],
                             vbuf[s], vsem[s])

        def stage(ci, s):
            pltpu.make_async_copy(verts.at[pl.ds(0, CHUNK)], vbuf[s],
                                  vsem[s]).wait()
            for g in range(CHUNK // LANES):
                sl = pl.ds(g * LANES, LANES)
                rows = g * LANES + iota
                x = plsc.load_gather(vbuf[s], [rows, zeros16])
                y = plsc.load_gather(vbuf[s], [rows, ones16])
                fx = jnp.minimum(jnp.maximum((x + 1.0) * 0.5 * (W - 1.0), 0.0),
                                 W - 1.0)
                fy = jnp.minimum(jnp.maximum((y + 1.0) * 0.5 * (H - 1.0), 0.0),
                                 H - 1.0)
                ix0 = fx.astype(jnp.int32)
                iy0 = fy.astype(jnp.int32)
                wx1 = fx - ix0.astype(jnp.float32)
                wy1 = fy - iy0.astype(jnp.float32)
                wx0 = 1.0 - wx1
                wy0 = 1.0 - wy1
                dx = jnp.where(ix0 < W - 1, 1, 0)
                dy = jnp.where(iy0 < H - 1, W, 0)
                p00 = iy0 * W + ix0
                i00[s][sl] = p00
                i01[s][sl] = p00 + dx
                i10[s][sl] = p00 + dy
                i11[s][sl] = p00 + dy + dx
                w00[s][sl] = wy0 * wx0
                w01[s][sl] = wy0 * wx1
                w10[s][sl] = wy1 * wx0
                w11[s][sl] = wy1 * wx1

            @pl.when(ci + NBUF < nch)
            def _():
                fetch_verts(ci + NBUF, s)
            pltpu.async_copy(table.at[i00[s]], v00[s], gsem[s])
            pltpu.async_copy(table.at[i01[s]], v01[s], gsem[s])
            pltpu.async_copy(table.at[i10[s]], v10[s], gsem[s])
            pltpu.async_copy(table.at[i11[s]], v11[s], gsem[s])

        def compute_write(ci, s):
            for buf in (v00, v01, v10, v11):
                pltpu.make_async_copy(table.at[i00[s]], buf[s], gsem[s]).wait()

            @pl.when(ci >= NBUF)
            def _():
                pltpu.make_async_copy(out.at[pl.ds(0, CHUNK)], outb[s],
                                      osem[s]).wait()

            def row_body(r2, carry2):
                for u in range(2):
                    r = r2 * 2 + u
                    bidx = zeros16 + r
                    b00 = plsc.load_gather(w00[s], [bidx])
                    b01 = plsc.load_gather(w01[s], [bidx])
                    b10 = plsc.load_gather(w10[s], [bidx])
                    b11 = plsc.load_gather(w11[s], [bidx])
                    for j in range(C // LANES):
                        sl = pl.ds(j * LANES, LANES)
                        outb[s][r, sl] = (v00[s][r, sl] * b00
                                          + v01[s][r, sl] * b01
                                          + v10[s][r, sl] * b10
                                          + v11[s][r, sl] * b11)
                return carry2

            lax.fori_loop(0, CHUNK // 2, row_body, 0)
            if TAIL:
                full = jnp.logical_not(
                    jnp.logical_and(wid == WTAIL, ci == KTAIL))
            else:
                full = wid == wid

            @pl.when(full)
            def _():
                pltpu.async_copy(outb[s], out.at[pl.ds(rbase(ci), CHUNK)],
                                 osem[s])

            if TAIL:
                @pl.when(jnp.logical_not(full))
                def _():
                    pltpu.sync_copy(outb[s].at[pl.ds(0, TAIL)],
                                    out.at[pl.ds(rbase(ci), TAIL)])

        fetch_verts(0, 0)
        fetch_verts(1, 1)
        stage(0, 0)

        def pair_body(k, carry):
            i0 = 2 * k
            i1 = i0 + 1

            @pl.when(i1 < nch)
            def _():
                stage(i1, 1)

            @pl.when(i0 < nch)
            def _():
                compute_write(i0, 0)

            @pl.when(i1 < nch)
            def _():
                @pl.when(i1 + 1 < nch)
                def _():
                    stage(i1 + 1, 0)
                compute_write(i1, 1)
            return carry

        lax.fori_loop(0, (NCH_LO + 2) // 2, pair_body, 0)

        for s in range(NBUF):
            def drain(s=s):
                pltpu.make_async_copy(out.at[pl.ds(0, CHUNK)], outb[s],
                                      osem[s]).wait()
            if TAIL and KTAIL % NBUF == s:
                pl.when(wid != WTAIL)(drain)
            else:
                drain()

    return sc_sample


def kernel(image_features, vertices):
    B, C, H, W = image_features.shape
    N = vertices.shape[1]
    sc_sample = _build_sc_call(C, H, W, N)
    vpad = jnp.pad(vertices, ((0, 0), (0, CHUNK), (0, 0)))
    outs = []
    for b in range(B):
        table_b = jnp.transpose(image_features[b], (1, 2, 0)).reshape(H * W, C)
        outs.append(sc_sample(table_b, vpad[b]))
    return jnp.stack(outs)


# speedup vs baseline: 1.5804x; 1.5804x over previous
import functools

import jax
import jax.numpy as jnp
from jax import lax
from jax.experimental import pallas as pl
from jax.experimental.pallas import tpu as pltpu
from jax.experimental.pallas import tpu_sc as plsc

NC = 2
NS = 16
NW = NC * NS
LANES = 16
CHUNK = 96
CP = 128
NBUF = 2


@functools.cache
def _build_sc_call(B, C, H, W, N):
    NPIX = H * W
    WPB = NW // B
    SPAN = ((N + WPB - 1) // WPB + CHUNK - 1) // CHUNK * CHUNK
    NFULL = SPAN // CHUNK
    LAST = N - (WPB - 1) * SPAN
    NFULL_LAST = LAST // CHUNK
    TAIL = LAST - NFULL_LAST * CHUNK
    NCH_LAST = NFULL_LAST + (1 if TAIL else 0)
    assert TAIL % LANES == 0 and LAST > 0

    mesh = plsc.VectorSubcoreMesh(core_axis_name="c", subcore_axis_name="s")

    def vmem(shape, dtype):
        return [pltpu.VMEM(shape, dtype) for _ in range(NBUF)]

    @functools.partial(
        pl.kernel,
        out_type=jax.ShapeDtypeStruct((B * N, C), jnp.float32),
        mesh=mesh,
        scratch_types=[
            vmem((CHUNK,), jnp.float32),
            vmem((CHUNK,), jnp.float32),
            vmem((CHUNK,), jnp.int32),
            vmem((CHUNK,), jnp.int32),
            vmem((CHUNK,), jnp.int32),
            vmem((CHUNK,), jnp.int32),
            vmem((CHUNK,), jnp.float32),
            vmem((CHUNK,), jnp.float32),
            vmem((CHUNK,), jnp.float32),
            vmem((CHUNK,), jnp.float32),
            vmem((CHUNK, CP), jnp.float32),
            vmem((CHUNK, CP), jnp.float32),
            vmem((CHUNK, CP), jnp.float32),
            vmem((CHUNK, CP), jnp.float32),
            vmem((CHUNK, C), jnp.float32),
            [pltpu.SemaphoreType.DMA for _ in range(NBUF)],
            [pltpu.SemaphoreType.DMA for _ in range(NBUF)],
            [pltpu.SemaphoreType.DMA for _ in range(NBUF)],
        ],
        compiler_params=pltpu.CompilerParams(needs_layout_passes=False,
                                             use_tc_tiling_on_sc=True),
    )
    def sc_sample(table, xs, ys, out, xbuf, ybuf, i00, i01, i10, i11,
                  w00, w01, w10, w11, v00, v01, v10, v11, outb,
                  vsem, gsem, osem):
        cid = lax.axis_index("c")
        sid = lax.axis_index("s")
        wid = sid * NC + cid
        wloc = lax.rem(wid, WPB)
        tab_off = lax.div(wid, WPB) * NPIX
        vbase = lax.div(wid, WPB) * N + wloc * SPAN
        is_last = wloc == WPB - 1
        nch = jnp.where(is_last, NCH_LAST, NFULL)

        iota = lax.broadcasted_iota(jnp.int32, (LANES,), 0)
        zeros16 = jnp.zeros((LANES,), jnp.int32)
        ones16 = zeros16 + 1

        def fetch_verts(ci, s):
            pltpu.async_copy(xs.at[pl.ds(vbase + ci * CHUNK, CHUNK)],
                             xbuf[s], vsem[s])
            pltpu.async_copy(ys.at[pl.ds(vbase + ci * CHUNK, CHUNK)],
                             ybuf[s], vsem[s])

        def stage(ci, s):
            pltpu.make_async_copy(xs.at[pl.ds(0, CHUNK)], xbuf[s],
                                  vsem[s]).wait()
            pltpu.make_async_copy(ys.at[pl.ds(0, CHUNK)], ybuf[s],
                                  vsem[s]).wait()
            for g in range(CHUNK // LANES):
                sl = pl.ds(g * LANES, LANES)
                x = xbuf[s][sl]
                y = ybuf[s][sl]
                fx = jnp.minimum(jnp.maximum((x + 1.0) * 0.5 * (W - 1.0), 0.0),
                                 W - 1.0)
                fy = jnp.minimum(jnp.maximum((y + 1.0) * 0.5 * (H - 1.0), 0.0),
                                 H - 1.0)
                ix0 = fx.astype(jnp.int32)
                iy0 = fy.astype(jnp.int32)
                wx1 = fx - ix0.astype(jnp.float32)
                wy1 = fy - iy0.astype(jnp.float32)
                wx0 = 1.0 - wx1
                wy0 = 1.0 - wy1
                dx = jnp.where(ix0 < W - 1, 1, 0)
                dy = jnp.where(iy0 < H - 1, W, 0)
                p00 = tab_off + iy0 * W + ix0
                i00[s][sl] = p00
                i01[s][sl] = p00 + dx
                i10[s][sl] = p00 + dy
                i11[s][sl] = p00 + dy + dx
                w00[s][sl] = wy0 * wx0
                w01[s][sl] = wy0 * wx1
                w10[s][sl] = wy1 * wx0
                w11[s][sl] = wy1 * wx1

            @pl.when(ci + NBUF < nch)
            def _():
                fetch_verts(ci + NBUF, s)
            pltpu.async_copy(table.at[i00[s]], v00[s], gsem[s])
            pltpu.async_copy(table.at[i01[s]], v01[s], gsem[s])
            pltpu.async_copy(table.at[i10[s]], v10[s], gsem[s])
            pltpu.async_copy(table.at[i11[s]], v11[s], gsem[s])

        def compute_write(ci, s):
            for buf in (v00, v01, v10, v11):
                pltpu.make_async_copy(table.at[i00[s]], buf[s], gsem[s]).wait()

            @pl.when(ci >= NBUF)
            def _():
                pltpu.make_async_copy(out.at[pl.ds(0, CHUNK)], outb[s],
                                      osem[s]).wait()

            def row_body(r2, carry2):
                for u in range(2):
                    r = r2 * 2 + u
                    bidx = zeros16 + r
                    b00 = plsc.load_gather(w00[s], [bidx])
                    b01 = plsc.load_gather(w01[s], [bidx])
                    b10 = plsc.load_gather(w10[s], [bidx])
                    b11 = plsc.load_gather(w11[s], [bidx])
                    for j in range(C // LANES):
                        sl = pl.ds(j * LANES, LANES)
                        outb[s][r, sl] = (v00[s][r, sl] * b00
                                          + v01[s][r, sl] * b01
                                          + v10[s][r, sl] * b10
                                          + v11[s][r, sl] * b11)
                return carry2

            lax.fori_loop(0, CHUNK // 2, row_body, 0)
            full = jnp.logical_or(jnp.logical_not(is_last), ci < NFULL_LAST)

            @pl.when(full)
            def _():
                pltpu.async_copy(outb[s],
                                 out.at[pl.ds(vbase + ci * CHUNK, CHUNK)],
                                 osem[s])

            if TAIL:
                @pl.when(jnp.logical_not(full))
                def _():
                    pltpu.sync_copy(
                        outb[s].at[pl.ds(0, TAIL)],
                        out.at[pl.ds(vbase + NFULL_LAST * CHUNK, TAIL)])

        fetch_verts(0, 0)
        fetch_verts(1, 1)
        stage(0, 0)

        def pair_body(k, carry):
            i0 = 2 * k
            i1 = i0 + 1

            @pl.when(i1 < nch)
            def _():
                stage(i1, 1)

            @pl.when(i0 < nch)
            def _():
                compute_write(i0, 0)

            @pl.when(i1 < nch)
            def _():
                @pl.when(i1 + 1 < nch)
                def _():
                    stage(i1 + 1, 0)
                compute_write(i1, 1)
            return carry

        lax.fori_loop(0, (max(NFULL, NCH_LAST) + 1) // 2, pair_body, 0)

        for s in range(NBUF):
            def drain(s=s):
                pltpu.make_async_copy(out.at[pl.ds(0, CHUNK)], outb[s],
                                      osem[s]).wait()
            if TAIL and (NCH_LAST - 1) % NBUF == s:
                pl.when(jnp.logical_not(is_last))(drain)
            else:
                drain()

    return sc_sample


def kernel(image_features, vertices):
    B, C, H, W = image_features.shape
    N = vertices.shape[1]
    table = jnp.pad(jnp.transpose(image_features, (0, 2, 3, 1)),
                    ((0, 0), (0, 0), (0, 0), (0, CP - C)))
    table = table.reshape(B * H * W, CP)
    pad = jnp.zeros((CHUNK,), vertices.dtype)
    xs = jnp.concatenate([vertices[..., 0].reshape(-1), pad])
    ys = jnp.concatenate([vertices[..., 1].reshape(-1), pad])
    sc_sample = _build_sc_call(B, C, H, W, N)
    out = sc_sample(table, xs, ys)
    return out.reshape(B, N, C)
